# Initial kernel scaffold; baseline (speedup 1.0000x reference)
import functools

import jax
import jax.numpy as jnp
from jax import lax
from jax.experimental import pallas as pl
from jax.experimental.pallas import tpu as pltpu
from jax.experimental.pallas import tpu_sc as plsc

N_NODES = 10000
N_PAD = 10240
E = 320000
D = 128
NC, NS, L = 2, 16, 16
NW = NC * NS
CHUNK = 640
SUB = 128
KSUB = CHUNK // SUB
NCHUNKS = E // CHUNK
ROWS_PER_TILE = N_PAD // NS


def _pre_body(x_ref, mu_ref, W1_ref, W3_ref, W4a_ref, W4c_ref,
              x1_ref, mu1_ref, h_ref):
    x1 = jax.nn.relu(x_ref[...] * W1_ref[...])
    mu1 = jax.nn.relu(jnp.dot(mu_ref[...], W3_ref[...],
                              preferred_element_type=jnp.float32))
    h = (jnp.dot(x1, W4a_ref[...], preferred_element_type=jnp.float32)
         + jnp.dot(mu1, W4c_ref[...], preferred_element_type=jnp.float32))
    x1_ref[...] = x1
    mu1_ref[...] = mu1
    h_ref[...] = h


def _pre_tc(x, mu, W1, W3, W4a, W4c):
    B = 2000
    grid = (N_NODES // B,)
    out = pl.pallas_call(
        _pre_body,
        grid=grid,
        in_specs=[
            pl.BlockSpec((B, 1), lambda i: (i, 0)),
            pl.BlockSpec((B, D), lambda i: (i, 0)),
            pl.BlockSpec((1, D), lambda i: (0, 0)),
            pl.BlockSpec((D, D), lambda i: (0, 0)),
            pl.BlockSpec((D, D), lambda i: (0, 0)),
            pl.BlockSpec((D, D), lambda i: (0, 0)),
        ],
        out_specs=[
            pl.BlockSpec((B, D), lambda i: (i, 0)),
            pl.BlockSpec((B, D), lambda i: (i, 0)),
            pl.BlockSpec((B, D), lambda i: (i, 0)),
        ],
        out_shape=[jax.ShapeDtypeStruct((N_NODES, D), jnp.float32)] * 3,
    )(x, mu, W1, W3, W4a, W4c)
    return out


def _sc_body(h_hbm, dst_hbm, src_hbm, ew_hbm,
             s0_hbm, s1_hbm, w0_hbm, w1_hbm,
             S_sp, SW_sp, rows_v, ew16_v, dst_v, src_v, ew_v, sem):
    cid_c = lax.axis_index("c")
    sid = lax.axis_index("s")
    wid = sid * NC + cid_c

    zro = jnp.zeros((L,), jnp.float32)

    def zrows(i, _):
        for j in range(D // L):
            rows_v[i, pl.ds(j * L, L)] = zro
        return 0
    lax.fori_loop(0, CHUNK, zrows, 0)

    def zew(i, _):
        ew16_v[i, :] = zro
        return 0
    lax.fori_loop(0, CHUNK, zew, 0)

    pltpu.sync_copy(rows_v, S_sp.at[pl.ds(sid * ROWS_PER_TILE, ROWS_PER_TILE)])
    pltpu.sync_copy(ew16_v, SW_sp.at[pl.ds(sid * ROWS_PER_TILE, ROWS_PER_TILE)])
    plsc.subcore_barrier()

    iota = lax.iota(jnp.int32, L)
    zidx = jnp.zeros((L,), jnp.int32)

    def chunk_body(i, _):
        cid = wid + i * NW

        @pl.when(cid < NCHUNKS)
        def _():
            row0 = cid * KSUB
            pltpu.sync_copy(dst_hbm.at[pl.ds(row0, KSUB)], dst_v)
            pltpu.sync_copy(src_hbm.at[pl.ds(row0, KSUB)], src_v)
            pltpu.sync_copy(ew_hbm.at[pl.ds(row0, KSUB)], ew_v)
            cps = []
            for j in range(KSUB):
                cps.append(pltpu.async_copy(
                    h_hbm.at[dst_v.at[j]],
                    rows_v.at[pl.ds(j * SUB, SUB)], sem))
            for cp in cps:
                cp.wait()
            for j in range(KSUB):
                def wr(b, _, j=j):
                    vals = ew_v[j, pl.ds(b * L, L)]
                    ridx = j * SUB + b * L + iota
                    plsc.store_scatter(ew16_v, [ridx, zidx], vals)
                    return 0
                lax.fori_loop(0, SUB // L, wr, 0)
            for j in range(KSUB):
                pltpu.sync_copy(rows_v.at[pl.ds(j * SUB, SUB)],
                                S_sp.at[src_v.at[j]], add=True)
                pltpu.sync_copy(ew16_v.at[pl.ds(j * SUB, SUB)],
                                SW_sp.at[src_v.at[j]], add=True)
        return 0

    lax.fori_loop(0, (NCHUNKS + NW - 1) // NW, chunk_body, 0)
    plsc.subcore_barrier()

    rsl = pl.ds(sid * ROWS_PER_TILE, ROWS_PER_TILE)

    @pl.when(cid_c == 0)
    def _():
        pltpu.sync_copy(S_sp.at[rsl], s0_hbm.at[rsl])
        pltpu.sync_copy(SW_sp.at[rsl], w0_hbm.at[rsl])

    @pl.when(cid_c == 1)
    def _():
        pltpu.sync_copy(S_sp.at[rsl], s1_hbm.at[rsl])
        pltpu.sync_copy(SW_sp.at[rsl], w1_hbm.at[rsl])


def _sc_scatter(h, dst2, src2, ew2):
    mesh = plsc.VectorSubcoreMesh(core_axis_name="c", subcore_axis_name="s")
    f32 = jnp.float32
    out_type = [
        jax.ShapeDtypeStruct((N_PAD, D), f32),
        jax.ShapeDtypeStruct((N_PAD, D), f32),
        jax.ShapeDtypeStruct((N_PAD, L), f32),
        jax.ShapeDtypeStruct((N_PAD, L), f32),
    ]
    scratch = [
        pltpu.VMEM_SHARED((N_PAD, D), f32),
        pltpu.VMEM_SHARED((N_PAD, L), f32),
        pltpu.VMEM((CHUNK, D), f32),
        pltpu.VMEM((CHUNK, L), f32),
        pltpu.VMEM((KSUB, SUB), jnp.int32),
        pltpu.VMEM((KSUB, SUB), jnp.int32),
        pltpu.VMEM((KSUB, SUB), f32),
        pltpu.SemaphoreType.DMA,
    ]
    fn = pl.kernel(_sc_body, out_type=out_type, mesh=mesh,
                   scratch_types=scratch)
    return fn(h, dst2, src2, ew2)


def _post_body(x1_ref, mu1_ref, S0_ref, S1_ref, sw0_ref, sw1_ref,
               W2_ref, W4b_ref, out_ref):
    v = jnp.dot(jax.nn.relu(W2_ref[...]), W4b_ref[...],
                preferred_element_type=jnp.float32)
    s = sw0_ref[...] + sw1_ref[...]
    pre = S0_ref[...] + S1_ref[...] + s * v
    out_ref[...] = jax.nn.relu(x1_ref[...] + mu1_ref[...] + jax.nn.relu(pre))


def _post_tc(x1, mu1, S0, S1, sw0, sw1, W2, W4b):
    B = 2000
    grid = (N_NODES // B,)
    return pl.pallas_call(
        _post_body,
        grid=grid,
        in_specs=[
            pl.BlockSpec((B, D), lambda i: (i, 0)),
            pl.BlockSpec((B, D), lambda i: (i, 0)),
            pl.BlockSpec((B, D), lambda i: (i, 0)),
            pl.BlockSpec((B, D), lambda i: (i, 0)),
            pl.BlockSpec((B, 1), lambda i: (i, 0)),
            pl.BlockSpec((B, 1), lambda i: (i, 0)),
            pl.BlockSpec((1, D), lambda i: (0, 0)),
            pl.BlockSpec((D, D), lambda i: (0, 0)),
        ],
        out_specs=pl.BlockSpec((B, D), lambda i: (i, 0)),
        out_shape=jax.ShapeDtypeStruct((N_NODES, D), jnp.float32),
    )(x1, mu1, S0, S1, sw0, sw1, W2, W4b)


def kernel(mu, x, edge_index, edge_w, W1, W2, W3, W4):
    src = edge_index[0].astype(jnp.int32).reshape(E // SUB, SUB)
    dst = edge_index[1].astype(jnp.int32).reshape(E // SUB, SUB)
    ew2 = edge_w.astype(jnp.float32).reshape(E // SUB, SUB)
    W4a, W4b, W4c = W4[:D], W4[D:2 * D], W4[2 * D:]

    x1, mu1, h = _pre_tc(x, mu, W1, W3, W4a, W4c)
    S0, S1, SW0, SW1 = _sc_scatter(h, dst, src, ew2)
    out = _post_tc(x1, mu1,
                   S0[:N_NODES], S1[:N_NODES],
                   SW0[:N_NODES, :1], SW1[:N_NODES, :1],
                   W2, W4b)
    return out


# speedup vs baseline: 8.8455x; 8.8455x over previous
import jax
import jax.numpy as jnp
from jax import lax
from jax.experimental import pallas as pl
from jax.experimental.pallas import tpu as pltpu
from jax.experimental.pallas import tpu_sc as plsc

N_NODES = 10000
E = 320000
D = 128
NC, NS, L = 2, 16, 16
HALF = 5120
JUNK = 128
NACC = HALF + JUNK
CHUNK = 512
SUB = 128
KSUB = CHUNK // SUB
NCHUNKS = E // CHUNK
ROWS_PER_TILE = NACC // NS


def _pre_body(x_ref, mu_ref, W1_ref, W3_ref, W4a_ref, W4c_ref,
              x1_ref, mu1_ref, h_ref):
    x1 = jax.nn.relu(x_ref[...] * W1_ref[...])
    mu1 = jax.nn.relu(jnp.dot(mu_ref[...], W3_ref[...],
                              preferred_element_type=jnp.float32))
    h = (jnp.dot(x1, W4a_ref[...], preferred_element_type=jnp.float32)
         + jnp.dot(mu1, W4c_ref[...], preferred_element_type=jnp.float32))
    x1_ref[...] = x1
    mu1_ref[...] = mu1
    h_ref[...] = h


def _pre_tc(x, mu, W1, W3, W4a, W4c):
    B = 2000
    grid = (N_NODES // B,)
    return pl.pallas_call(
        _pre_body,
        grid=grid,
        in_specs=[
            pl.BlockSpec((B, 1), lambda i: (i, 0)),
            pl.BlockSpec((B, D), lambda i: (i, 0)),
            pl.BlockSpec((1, D), lambda i: (0, 0)),
            pl.BlockSpec((D, D), lambda i: (0, 0)),
            pl.BlockSpec((D, D), lambda i: (0, 0)),
            pl.BlockSpec((D, D), lambda i: (0, 0)),
        ],
        out_specs=[
            pl.BlockSpec((B, D), lambda i: (i, 0)),
            pl.BlockSpec((B, D), lambda i: (i, 0)),
            pl.BlockSpec((B, D), lambda i: (i, 0)),
        ],
        out_shape=[jax.ShapeDtypeStruct((N_NODES, D), jnp.float32)] * 3,
    )(x, mu, W1, W3, W4a, W4c)


def _sc_body(h_hbm, dst_hbm, src_hbm, ew_hbm, zrow_hbm, zw_hbm,
             s0_hbm, s1_hbm, w0_hbm, w1_hbm,
             S_sp, SW_sp, rows_v, dst_v, src_v, lsrc_v, ew_v, sem):
    cid_c = lax.axis_index("c")
    sid = lax.axis_index("s")
    base = cid_c * HALF

    rsl = pl.ds(sid * ROWS_PER_TILE, ROWS_PER_TILE)
    pltpu.sync_copy(zrow_hbm.at[rsl], S_sp.at[rsl])

    @pl.when(sid == 0)
    def _():
        pltpu.sync_copy(zw_hbm, SW_sp)

    plsc.subcore_barrier()

    def chunk_body(i, _):
        cid = sid + i * NS

        @pl.when(cid < NCHUNKS)
        def _():
            pltpu.sync_copy(dst_hbm.at[cid], dst_v)
            pltpu.sync_copy(src_hbm.at[cid], src_v)
            pltpu.sync_copy(ew_hbm.at[cid], ew_v)
            cps = []
            for j in range(KSUB):
                cps.append(pltpu.async_copy(
                    h_hbm.at[dst_v.at[j]],
                    rows_v.at[pl.ds(j * SUB, SUB)], sem))
            for j in range(KSUB):
                for b in range(SUB // L):
                    vec = src_v[j, pl.ds(b * L, L)]
                    loc = vec - base
                    ok = (loc >= 0) & (loc < HALF)
                    idx = jnp.where(ok, loc, HALF + (vec & (JUNK - 1)))
                    lsrc_v[j, pl.ds(b * L, L)] = idx
            for cp in cps:
                cp.wait()
            for j in range(KSUB):
                pltpu.sync_copy(rows_v.at[pl.ds(j * SUB, SUB)],
                                S_sp.at[lsrc_v.at[j]], add=True)
                pltpu.sync_copy(ew_v.at[j],
                                SW_sp.at[lsrc_v.at[j]], add=True)
        return 0

    lax.fori_loop(0, (NCHUNKS + NS - 1) // NS, chunk_body, 0)
    plsc.subcore_barrier()

    @pl.when(cid_c == 0)
    def _():
        pltpu.sync_copy(S_sp.at[rsl], s0_hbm.at[rsl])

        @pl.when(sid == 0)
        def _():
            pltpu.sync_copy(SW_sp, w0_hbm)

    @pl.when(cid_c == 1)
    def _():
        pltpu.sync_copy(S_sp.at[rsl], s1_hbm.at[rsl])

        @pl.when(sid == 0)
        def _():
            pltpu.sync_copy(SW_sp, w1_hbm)


def _sc_scatter(h, dst2, src2, ew2, zrow, zw):
    mesh = plsc.VectorSubcoreMesh(core_axis_name="c", subcore_axis_name="s")
    f32 = jnp.float32
    out_type = [
        jax.ShapeDtypeStruct((NACC, D), f32),
        jax.ShapeDtypeStruct((NACC, D), f32),
        jax.ShapeDtypeStruct((NACC,), f32),
        jax.ShapeDtypeStruct((NACC,), f32),
    ]
    scratch = [
        pltpu.VMEM_SHARED((NACC, D), f32),
        pltpu.VMEM_SHARED((NACC,), f32),
        pltpu.VMEM((CHUNK, D), f32),
        pltpu.VMEM((KSUB, SUB), jnp.int32),
        pltpu.VMEM((KSUB, SUB), jnp.int32),
        pltpu.VMEM((KSUB, SUB), jnp.int32),
        pltpu.VMEM((KSUB, SUB), f32),
        pltpu.SemaphoreType.DMA,
    ]
    fn = pl.kernel(_sc_body, out_type=out_type, mesh=mesh,
                   scratch_types=scratch)
    return fn(h, dst2, src2, ew2, zrow, zw)


def _post_body(x1_ref, mu1_ref, S_ref, sw_ref, W2_ref, W4b_ref, out_ref):
    v = jnp.dot(jax.nn.relu(W2_ref[...]), W4b_ref[...],
                preferred_element_type=jnp.float32)
    pre = S_ref[...] + sw_ref[...] * v
    out_ref[...] = jax.nn.relu(x1_ref[...] + mu1_ref[...] + jax.nn.relu(pre))


def _post_tc(x1, mu1, S, sw, W2, W4b):
    B = 2000
    grid = (N_NODES // B,)
    return pl.pallas_call(
        _post_body,
        grid=grid,
        in_specs=[
            pl.BlockSpec((B, D), lambda i: (i, 0)),
            pl.BlockSpec((B, D), lambda i: (i, 0)),
            pl.BlockSpec((B, D), lambda i: (i, 0)),
            pl.BlockSpec((B, 1), lambda i: (i, 0)),
            pl.BlockSpec((1, D), lambda i: (0, 0)),
            pl.BlockSpec((D, D), lambda i: (0, 0)),
        ],
        out_specs=pl.BlockSpec((B, D), lambda i: (i, 0)),
        out_shape=jax.ShapeDtypeStruct((N_NODES, D), jnp.float32),
    )(x1, mu1, S, sw, W2, W4b)


def kernel(mu, x, edge_index, edge_w, W1, W2, W3, W4):
    src = edge_index[0].astype(jnp.int32).reshape(NCHUNKS, KSUB, SUB)
    dst = edge_index[1].astype(jnp.int32).reshape(NCHUNKS, KSUB, SUB)
    ew2 = edge_w.astype(jnp.float32).reshape(NCHUNKS, KSUB, SUB)
    W4a, W4b, W4c = W4[:D], W4[D:2 * D], W4[2 * D:]

    x1, mu1, h = _pre_tc(x, mu, W1, W3, W4a, W4c)
    zrow = jnp.zeros((NACC, D), jnp.float32)
    zw = jnp.zeros((NACC,), jnp.float32)
    S0, S1, SW0, SW1 = _sc_scatter(h, dst, src, ew2, zrow, zw)
    S = jnp.concatenate([S0[:HALF], S1[:N_NODES - HALF]], axis=0)
    sw = jnp.concatenate([SW0[:HALF], SW1[:N_NODES - HALF]])[:, None]
    out = _post_tc(x1, mu1, S, sw, W2, W4b)
    return out


# speedup vs baseline: 12.2459x; 1.3844x over previous
import jax
import jax.numpy as jnp
from jax import lax
from jax.experimental import pallas as pl
from jax.experimental.pallas import tpu as pltpu
from jax.experimental.pallas import tpu_sc as plsc

N_NODES = 10000
N_PAD = 10240
E = 320000
D = 128
NC, NS, L = 2, 16, 16
NW = NC * NS
CHUNK = 256
SUB = 128
KSUB = CHUNK // SUB
NCHUNKS = E // CHUNK
ROWS_PER_TILE = N_PAD // NS


def _pre_body(x_ref, mu_ref, W1_ref, W3_ref, W4a_ref, W4c_ref,
              x1_ref, mu1_ref, h_ref):
    x1 = jax.nn.relu(x_ref[...] * W1_ref[...])
    mu1 = jax.nn.relu(jnp.dot(mu_ref[...], W3_ref[...],
                              preferred_element_type=jnp.float32))
    h = (jnp.dot(x1, W4a_ref[...], preferred_element_type=jnp.float32)
         + jnp.dot(mu1, W4c_ref[...], preferred_element_type=jnp.float32))
    x1_ref[...] = x1
    mu1_ref[...] = mu1
    h_ref[...] = h


def _pre_tc(x, mu, W1, W3, W4a, W4c):
    B = 2000
    grid = (N_NODES // B,)
    return pl.pallas_call(
        _pre_body,
        grid=grid,
        in_specs=[
            pl.BlockSpec((B, 1), lambda i: (i, 0)),
            pl.BlockSpec((B, D), lambda i: (i, 0)),
            pl.BlockSpec((1, D), lambda i: (0, 0)),
            pl.BlockSpec((D, D), lambda i: (0, 0)),
            pl.BlockSpec((D, D), lambda i: (0, 0)),
            pl.BlockSpec((D, D), lambda i: (0, 0)),
        ],
        out_specs=[
            pl.BlockSpec((B, D), lambda i: (i, 0)),
            pl.BlockSpec((B, D), lambda i: (i, 0)),
            pl.BlockSpec((B, D), lambda i: (i, 0)),
        ],
        out_shape=[jax.ShapeDtypeStruct((N_NODES, D), jnp.float32)] * 3,
    )(x, mu, W1, W3, W4a, W4c)


def _sc_body(h_hbm, dst_hbm, src_hbm, ew_hbm, zrow_hbm, zw_hbm,
             s0_hbm, s1_hbm, w0_hbm, w1_hbm,
             S_sp, SW_sp, rows_v, dst_v, src_v, ew_v, sem):
    cid_c = lax.axis_index("c")
    sid = lax.axis_index("s")
    wid = sid * NC + cid_c

    rsl = pl.ds(sid * ROWS_PER_TILE, ROWS_PER_TILE)
    pltpu.sync_copy(zrow_hbm.at[rsl], S_sp.at[rsl])

    @pl.when(sid == 0)
    def _():
        pltpu.sync_copy(zw_hbm, SW_sp)

    plsc.subcore_barrier()

    def chunk_body(i, _):
        cid = wid + i * NW

        @pl.when(cid < NCHUNKS)
        def _():
            pltpu.sync_copy(dst_hbm.at[cid], dst_v)
            pltpu.sync_copy(src_hbm.at[cid], src_v)
            pltpu.sync_copy(ew_hbm.at[cid], ew_v)
            cps = []
            for j in range(KSUB):
                cps.append(pltpu.async_copy(
                    h_hbm.at[dst_v.at[j]],
                    rows_v.at[pl.ds(j * SUB, SUB)], sem))
            for cp in cps:
                cp.wait()
            for j in range(KSUB):
                pltpu.sync_copy(rows_v.at[pl.ds(j * SUB, SUB)],
                                S_sp.at[src_v.at[j]], add=True)
                pltpu.sync_copy(ew_v.at[j],
                                SW_sp.at[src_v.at[j]], add=True)
        return 0

    lax.fori_loop(0, (NCHUNKS + NW - 1) // NW, chunk_body, 0)
    plsc.subcore_barrier()

    @pl.when(cid_c == 0)
    def _():
        pltpu.sync_copy(S_sp.at[rsl], s0_hbm.at[rsl])

        @pl.when(sid == 0)
        def _():
            pltpu.sync_copy(SW_sp, w0_hbm)

    @pl.when(cid_c == 1)
    def _():
        pltpu.sync_copy(S_sp.at[rsl], s1_hbm.at[rsl])

        @pl.when(sid == 0)
        def _():
            pltpu.sync_copy(SW_sp, w1_hbm)


def _sc_scatter(h, dst2, src2, ew2, zrow, zw):
    mesh = plsc.VectorSubcoreMesh(core_axis_name="c", subcore_axis_name="s")
    f32 = jnp.float32
    out_type = [
        jax.ShapeDtypeStruct((N_PAD, D), f32),
        jax.ShapeDtypeStruct((N_PAD, D), f32),
        jax.ShapeDtypeStruct((N_PAD,), f32),
        jax.ShapeDtypeStruct((N_PAD,), f32),
    ]
    scratch = [
        pltpu.VMEM_SHARED((N_PAD, D), f32),
        pltpu.VMEM_SHARED((N_PAD,), f32),
        pltpu.VMEM((CHUNK, D), f32),
        pltpu.VMEM((KSUB, SUB), jnp.int32),
        pltpu.VMEM((KSUB, SUB), jnp.int32),
        pltpu.VMEM((KSUB, SUB), f32),
        pltpu.SemaphoreType.DMA,
    ]
    fn = pl.kernel(_sc_body, out_type=out_type, mesh=mesh,
                   scratch_types=scratch)
    return fn(h, dst2, src2, ew2, zrow, zw)


def _post_body(x1_ref, mu1_ref, S0_ref, S1_ref, sw0_ref, sw1_ref,
               W2_ref, W4b_ref, out_ref):
    v = jnp.dot(jax.nn.relu(W2_ref[...]), W4b_ref[...],
                preferred_element_type=jnp.float32)
    pre = S0_ref[...] + S1_ref[...] + (sw0_ref[...] + sw1_ref[...]) * v
    out_ref[...] = jax.nn.relu(x1_ref[...] + mu1_ref[...] + jax.nn.relu(pre))


def _post_tc(x1, mu1, S0, S1, sw0, sw1, W2, W4b):
    B = 2000
    grid = (N_NODES // B,)
    return pl.pallas_call(
        _post_body,
        grid=grid,
        in_specs=[
            pl.BlockSpec((B, D), lambda i: (i, 0)),
            pl.BlockSpec((B, D), lambda i: (i, 0)),
            pl.BlockSpec((B, D), lambda i: (i, 0)),
            pl.BlockSpec((B, D), lambda i: (i, 0)),
            pl.BlockSpec((B, 1), lambda i: (i, 0)),
            pl.BlockSpec((B, 1), lambda i: (i, 0)),
            pl.BlockSpec((1, D), lambda i: (0, 0)),
            pl.BlockSpec((D, D), lambda i: (0, 0)),
        ],
        out_specs=pl.BlockSpec((B, D), lambda i: (i, 0)),
        out_shape=jax.ShapeDtypeStruct((N_NODES, D), jnp.float32),
    )(x1, mu1, S0, S1, sw0, sw1, W2, W4b)


def kernel(mu, x, edge_index, edge_w, W1, W2, W3, W4):
    src = edge_index[0].astype(jnp.int32).reshape(NCHUNKS, KSUB, SUB)
    dst = edge_index[1].astype(jnp.int32).reshape(NCHUNKS, KSUB, SUB)
    ew2 = edge_w.astype(jnp.float32).reshape(NCHUNKS, KSUB, SUB)
    W4a, W4b, W4c = W4[:D], W4[D:2 * D], W4[2 * D:]

    x1, mu1, h = _pre_tc(x, mu, W1, W3, W4a, W4c)
    zrow = jnp.zeros((N_PAD, D), jnp.float32)
    zw = jnp.zeros((N_PAD,), jnp.float32)
    S0, S1, SW0, SW1 = _sc_scatter(h, dst, src, ew2, zrow, zw)
    out = _post_tc(x1, mu1, S0[:N_NODES], S1[:N_NODES],
                   SW0[:N_NODES, None], SW1[:N_NODES, None], W2, W4b)
    return out
